# baseline (speedup 1.0000x reference)
import functools

import jax
import jax.numpy as jnp
from jax import lax
from jax.experimental import pallas as pl
from jax.experimental.pallas import tpu as pltpu
from jax.experimental.pallas import tpu_sc as plsc

_IOU_THRESHOLD = 0.6
_K = 50
_LANES = 128
_RB = 8
_NPAD = 5120
_ROWS = _NPAD // _LANES
_NSETS = 4

_TC_N = 4096
_TC_NBLK = _TC_N // (_RB * _LANES)

_SC_N = _NPAD - _TC_N
_NCHUNK = 8
_CHUNK = _SC_N // _NCHUNK
_NV = _CHUNK // 16
_NP = 32


def _clip_dt(pu, pv, du, dv, rxu, rxv, hx, hy):
    ta = -(hx + pu) * rxu
    tb = (hx - pu) * rxu
    lo_u = jnp.minimum(ta, tb)
    hi_u = jnp.maximum(ta, tb)
    tc = -(hy + pv) * rxv
    td = (hy - pv) * rxv
    lo_v = jnp.minimum(tc, td)
    hi_v = jnp.maximum(tc, td)
    t0 = jnp.maximum(jnp.maximum(lo_u, lo_v), 0.0)
    t1 = jnp.minimum(jnp.minimum(hi_u, hi_v), 1.0)
    return jnp.maximum(t1 - t0, 0.0)


def _pair_step(p, g, k, best, bki):
    (pax0, pax1, pax2, pax3, pay0, pay1, pay2, pay3,
     e0x, e0y, e1x, e1y, x, y, ca, sa, hxa, hya, za1, za2, va) = p
    (cx, cy, cb, sb, hx, hy, zb1, zb2, vb,
     qx0, qy0, qx1, qy1, qx2, qy2, qx3, qy3, d0x, d0y, d1x, d1y) = g

    us = []
    vs = []
    for pax, pay in ((pax0, pay0), (pax1, pay1), (pax2, pay2), (pax3, pay3)):
        tx = pax - cx
        ty = pay - cy
        us.append(cb * tx + sb * ty)
        vs.append(cb * ty - sb * tx)
    e0u = cb * e0x + sb * e0y
    e0v = cb * e0y - sb * e0x
    e1u = cb * e1x + sb * e1y
    e1v = cb * e1y - sb * e1x
    r0u = 1.0 / e0u
    r0v = 1.0 / e0v
    r1u = 1.0 / e1u
    r1v = 1.0 / e1v
    eds = ((e0u, e0v, r0u, r0v), (e1u, e1v, r1u, r1v),
           (-e0u, -e0v, -r0u, -r0v), (-e1u, -e1v, -r1u, -r1v))
    area2 = jnp.zeros_like(x)
    for j in range(4):
        du, dv, ru, rv = eds[j]
        dt = _clip_dt(us[j], vs[j], du, dv, ru, rv, hx, hy)
        area2 = area2 + dt * (us[j] * dv - vs[j] * du)

    qs = []
    for qxj, qyj in ((qx0, qy0), (qx1, qy1), (qx2, qy2), (qx3, qy3)):
        tx = qxj - x
        ty = qyj - y
        qs.append((ca * tx + sa * ty, ca * ty - sa * tx))
    d0u = ca * d0x + sa * d0y
    d0v = ca * d0y - sa * d0x
    d1u = ca * d1x + sa * d1y
    d1v = ca * d1y - sa * d1x
    s0u = 1.0 / d0u
    s0v = 1.0 / d0v
    s1u = 1.0 / d1u
    s1v = 1.0 / d1v
    bds = ((d0u, d0v, s0u, s0v), (d1u, d1v, s1u, s1v),
           (-d0u, -d0v, -s0u, -s0v), (-d1u, -d1v, -s1u, -s1v))
    dtsum = jnp.zeros_like(x)
    for j in range(4):
        du, dv, ru, rv = bds[j]
        dtsum = dtsum + _clip_dt(qs[j][0], qs[j][1], du, dv, ru, rv, hxa, hya)
    area2 = area2 + dtsum * (2.0 * hx * hy)

    area = jnp.maximum(0.5 * area2, 0.0)
    h = jnp.maximum(jnp.minimum(za2, zb2) - jnp.maximum(za1, zb1), 0.0)
    inter = area * h
    iou = inter / jnp.maximum(va + vb - inter, 1e-6)
    upd = iou > best
    best = jnp.where(upd, iou, best)
    bki = jnp.where(upd, jnp.broadcast_to(k, bki.shape), bki)
    return best, bki


def _derive(x, y, z, dxa, dya, dza, r):
    ca = jnp.cos(r)
    sa = jnp.sin(r)
    hxa = 0.5 * dxa
    hya = 0.5 * dya
    cxh = ca * hxa
    sxh = sa * hxa
    cyh = ca * hya
    syh = sa * hya
    pax = (x + cxh - syh, x - cxh - syh, x - cxh + syh, x + cxh + syh)
    pay = (y + sxh + cyh, y - sxh + cyh, y - sxh - cyh, y + sxh - cyh)
    e = (-2.0 * cxh, -2.0 * sxh, 2.0 * syh, -2.0 * cyh)
    za1 = z - 0.5 * dza
    za2 = z + 0.5 * dza
    va = dxa * dya * dza
    return pax, pay, e, ca, sa, hxa, hya, za1, za2, va


def _tc_assign_body(pred_ref, gt_ref, out_ref):
    x = pred_ref[0, 0]
    y = pred_ref[0, 1]
    pax, pay, e, ca, sa, hxa, hya, za1, za2, va = _derive(
        x, y, pred_ref[0, 2], pred_ref[0, 3], pred_ref[0, 4],
        pred_ref[0, 5], pred_ref[0, 6])
    p = (pax[0], pax[1], pax[2], pax[3], pay[0], pay[1], pay[2], pay[3],
         e[0], e[1], e[2], e[3], x, y, ca, sa, hxa, hya, za1, za2, va)

    shape = x.shape
    best0 = jnp.full(shape, -1.0, jnp.float32)
    bki0 = jnp.zeros(shape, jnp.int32)

    def step(k, carry):
        best, bki = carry
        g = tuple(gt_ref[0, k, j] for j in range(21))
        return _pair_step(p, g, k, best, bki)

    best, bki = jax.lax.fori_loop(0, _K, step, (best0, bki0), unroll=50)
    out_ref[0] = jnp.where(best < _IOU_THRESHOLD, -1, bki)


def _prep_body(pred_ref, stat_ref, out_ref):
    x = pred_ref[0, 0]
    y = pred_ref[0, 1]
    pax, pay, e, ca, sa, hxa, hya, za1, za2, va = _derive(
        x, y, pred_ref[0, 2], pred_ref[0, 3], pred_ref[0, 4],
        pred_ref[0, 5], pred_ref[0, 6])
    for j in range(4):
        out_ref[0, j] = pax[j]
        out_ref[0, 4 + j] = pay[j]
        out_ref[0, 8 + j] = e[j]
    out_ref[0, 12] = x
    out_ref[0, 13] = y
    out_ref[0, 14] = ca
    out_ref[0, 15] = sa
    out_ref[0, 16] = hxa
    out_ref[0, 17] = hya
    out_ref[0, 18] = za1
    out_ref[0, 19] = za2
    out_ref[0, 20] = va
    out_ref[0, 21] = jnp.sqrt(hxa * hxa + hya * hya)
    out_ref[0, 22] = stat_ref[0, 0]
    out_ref[0, 23] = stat_ref[0, 1]
    out_ref[0, 24] = stat_ref[0, 2]
    zero = jnp.zeros_like(x)
    for j in range(25, _NP):
        out_ref[0, j] = zero


def _sc_assign_body(planes_hbm, gt_hbm, out_hbm, pv, gv, ov, bv, kv):
    cid = lax.axis_index("c")
    sid = lax.axis_index("s")
    wid = sid * 2 + cid
    st = wid // _NCHUNK
    ck = lax.rem(wid, _NCHUNK)
    pltpu.sync_copy(planes_hbm.at[st, ck], pv)
    pltpu.sync_copy(gt_hbm.at[st], gv)

    def outer(v, carry_o):
        off = v * 16
        p = tuple(pv[pl.ds(j * _CHUNK + off, 16)] for j in range(21))
        xlo = pv[pl.ds(22 * _CHUNK + off, 16)][0]
        xhi = pv[pl.ds(23 * _CHUNK + off, 16)][0]
        rmx = pv[pl.ds(24 * _CHUNK + off, 16)][0]
        xc = 0.5 * (xlo + xhi)
        xh = 0.5 * (xhi - xlo) + rmx
        bv[pl.ds(0, 16)] = jnp.full((16,), -1.0, jnp.float32)
        kv[pl.ds(0, 16)] = jnp.zeros((16,), jnp.int32)

        def inner(k, carry):
            cxs = gv[pl.ds(k * 384 + 0 * 16, 16)][0]
            rbs = gv[pl.ds(k * 384 + 21 * 16, 16)][0]
            d = cxs - xc

            @pl.when(jnp.abs(d) < xh + rbs)
            def _():
                g = tuple(gv[pl.ds(k * 384 + j * 16, 16)] for j in range(21))
                best, bki = _pair_step(p, g, k, bv[pl.ds(0, 16)],
                                       kv[pl.ds(0, 16)])
                bv[pl.ds(0, 16)] = best
                kv[pl.ds(0, 16)] = bki

            return carry

        lax.fori_loop(0, _K, inner, 0)
        best = bv[pl.ds(0, 16)]
        ov[pl.ds(off, 16)] = jnp.where(best < _IOU_THRESHOLD,
                                       jnp.full((16,), -1, jnp.int32),
                                       kv[pl.ds(0, 16)])
        return carry_o

    lax.fori_loop(0, _NV, outer, 0)
    pltpu.sync_copy(ov, out_hbm.at[st, ck])


def _gt_scalars(gt):
    cx = gt[..., 0]
    cy = gt[..., 1]
    zc = gt[..., 2]
    dx = gt[..., 3]
    dy = gt[..., 4]
    dz = gt[..., 5]
    rr = gt[..., 6]
    cb = jnp.cos(rr)
    sb = jnp.sin(rr)
    hx = 0.5 * dx
    hy = 0.5 * dy
    cxh = cb * hx
    sxh = sb * hx
    cyh = cb * hy
    syh = sb * hy
    qx0 = cx + cxh - syh
    qy0 = cy + sxh + cyh
    qx1 = cx - cxh - syh
    qy1 = cy - sxh + cyh
    qx2 = cx - cxh + syh
    qy2 = cy - sxh - cyh
    qx3 = cx + cxh + syh
    qy3 = cy + sxh - cyh
    d0x = qx1 - qx0
    d0y = qy1 - qy0
    d1x = qx2 - qx1
    d1y = qy2 - qy1
    zb1 = zc - 0.5 * dz
    zb2 = zc + 0.5 * dz
    vb = dx * dy * dz
    rb = jnp.sqrt(hx * hx + hy * hy)
    pad = jnp.zeros_like(cx)
    return jnp.stack([cx, cy, cb, sb, hx, hy, zb1, zb2, vb,
                      qx0, qy0, qx1, qy1, qx2, qy2, qx3, qy3,
                      d0x, d0y, d1x, d1y, rb, pad, pad], axis=-1)


def kernel(teacher_boxes, student_boxes, gt_boxes):
    B, N, _ = teacher_boxes.shape
    pred = jnp.concatenate([teacher_boxes, student_boxes], axis=0)
    padbox = jnp.zeros((2 * B, _NPAD - N, 7), jnp.float32).at[:, :, 3:6].set(1.0)
    pred = jnp.concatenate([pred, padbox], axis=1)

    gtp = _gt_scalars(gt_boxes)
    gtp = jnp.tile(gtp, (2, 1, 1))
    gtb = jnp.broadcast_to(gtp[..., None], (_NSETS, _K, 24, 16))
    gtb = gtb.reshape(_NSETS, _K * 24 * 16)

    preds = pred[:, _TC_N:]
    perm = jnp.argsort(preds[:, :, 0], axis=-1)
    preds = jnp.take_along_axis(preds, perm[:, :, None], axis=1)
    planes_s = jnp.transpose(preds, (0, 2, 1))
    planes_s = jnp.concatenate(
        [planes_s, jnp.zeros((_NSETS, 1, _SC_N), jnp.float32)], axis=1)
    rows_s = _SC_N // _LANES
    planes_s = planes_s.reshape(_NSETS, 8, rows_s, _LANES)

    xg = preds[:, :, 0].reshape(_NSETS, _SC_N // 16, 16)
    rg = 0.5 * jnp.sqrt(preds[:, :, 3] ** 2 + preds[:, :, 4] ** 2)
    rg = rg.reshape(_NSETS, _SC_N // 16, 16)
    stats = jnp.stack([xg.min(-1), xg.max(-1), rg.max(-1)], axis=1)
    stats = jnp.broadcast_to(stats[..., None], (_NSETS, 3, _SC_N // 16, 16))
    stats = stats.reshape(_NSETS, 3, rows_s, _LANES)
    stats = jnp.concatenate(
        [stats, jnp.zeros((_NSETS, 5, rows_s, _LANES), jnp.float32)], axis=1)

    derived = pl.pallas_call(
        _prep_body,
        grid=(_NSETS, rows_s // _RB),
        in_specs=[
            pl.BlockSpec((1, 8, _RB, _LANES), lambda c, nb: (c, 0, nb, 0)),
            pl.BlockSpec((1, 8, _RB, _LANES), lambda c, nb: (c, 0, nb, 0)),
        ],
        out_specs=pl.BlockSpec((1, _NP, _RB, _LANES), lambda c, nb: (c, 0, nb, 0)),
        out_shape=jax.ShapeDtypeStruct((_NSETS, _NP, rows_s, _LANES),
                                       jnp.float32),
    )(planes_s, stats)
    derived = derived.reshape(_NSETS, _NP, _CHUNK // 16, _NCHUNK, 16)
    derived = derived.transpose(0, 3, 1, 2, 4)
    derived = derived.reshape(_NSETS, _NCHUNK, _NP * _CHUNK)

    sc = functools.partial(
        pl.kernel,
        out_type=jax.ShapeDtypeStruct((_NSETS, _NCHUNK, _CHUNK), jnp.int32),
        mesh=plsc.VectorSubcoreMesh(core_axis_name="c", subcore_axis_name="s"),
        scratch_types=[
            pltpu.VMEM((_NP * _CHUNK,), jnp.float32),
            pltpu.VMEM((_K * 24 * 16,), jnp.float32),
            pltpu.VMEM((_CHUNK,), jnp.int32),
            pltpu.VMEM((16,), jnp.float32),
            pltpu.VMEM((16,), jnp.int32),
        ],
    )(_sc_assign_body)
    out_s = sc(derived, gtb)
    out_s = out_s.reshape(_NSETS, _NCHUNK, _CHUNK // 16, 16)
    out_s = out_s.transpose(0, 2, 1, 3).reshape(_NSETS, _SC_N)
    out_s = jnp.zeros_like(out_s).at[
        jnp.arange(_NSETS)[:, None], perm].set(out_s)

    planes_t = jnp.transpose(pred[:, :_TC_N], (0, 2, 1))
    planes_t = jnp.concatenate(
        [planes_t, jnp.zeros((_NSETS, 1, _TC_N), jnp.float32)], axis=1)
    planes_t = planes_t.reshape(_NSETS, 8, _TC_N // _LANES, _LANES)

    out_t = pl.pallas_call(
        _tc_assign_body,
        grid=(_NSETS, _TC_NBLK),
        in_specs=[
            pl.BlockSpec((1, 8, _RB, _LANES), lambda c, nb: (c, 0, nb, 0)),
            pl.BlockSpec((1, _K, 24), lambda c, nb: (c, 0, 0),
                         memory_space=pltpu.SMEM),
        ],
        out_specs=pl.BlockSpec((1, _RB, _LANES), lambda c, nb: (c, nb, 0)),
        out_shape=jax.ShapeDtypeStruct((_NSETS, _TC_N // _LANES, _LANES),
                                       jnp.int32),
    )(planes_t, gtp)
    out_t = out_t.reshape(_NSETS, _TC_N)

    out = jnp.concatenate([out_t, out_s], axis=1)[:, :N]
    return out[:B], out[B:]


# speedup vs baseline: 1.1078x; 1.1078x over previous
import functools

import jax
import jax.numpy as jnp
from jax import lax
from jax.experimental import pallas as pl
from jax.experimental.pallas import tpu as pltpu
from jax.experimental.pallas import tpu_sc as plsc

_IOU_THRESHOLD = 0.6
_K = 50
_LANES = 128
_RB = 8
_NPAD = 5120
_ROWS = _NPAD // _LANES
_NSETS = 4

_TC_N = 4096
_TC_NBLK = _TC_N // (_RB * _LANES)

_SC_N = _NPAD - _TC_N
_NCHUNK = 8
_CHUNK = _SC_N // _NCHUNK
_NV = _CHUNK // 16
_NP = 32


def _clip_dt(pu, pv, du, dv, rxu, rxv, hx, hy):
    ta = -(hx + pu) * rxu
    tb = (hx - pu) * rxu
    lo_u = jnp.minimum(ta, tb)
    hi_u = jnp.maximum(ta, tb)
    tc = -(hy + pv) * rxv
    td = (hy - pv) * rxv
    lo_v = jnp.minimum(tc, td)
    hi_v = jnp.maximum(tc, td)
    t0 = jnp.maximum(jnp.maximum(lo_u, lo_v), 0.0)
    t1 = jnp.minimum(jnp.minimum(hi_u, hi_v), 1.0)
    return jnp.maximum(t1 - t0, 0.0)


def _pair_step(p, g, k, best, bki):
    (pax0, pax1, pax2, pax3, pay0, pay1, pay2, pay3,
     e0x, e0y, e1x, e1y, x, y, ca, sa, hxa, hya, za1, za2, va) = p
    (cx, cy, cb, sb, hx, hy, zb1, zb2, vb,
     qx0, qy0, qx1, qy1, qx2, qy2, qx3, qy3, d0x, d0y, d1x, d1y) = g

    us = []
    vs = []
    for pax, pay in ((pax0, pay0), (pax1, pay1), (pax2, pay2), (pax3, pay3)):
        tx = pax - cx
        ty = pay - cy
        us.append(cb * tx + sb * ty)
        vs.append(cb * ty - sb * tx)
    e0u = cb * e0x + sb * e0y
    e0v = cb * e0y - sb * e0x
    e1u = cb * e1x + sb * e1y
    e1v = cb * e1y - sb * e1x
    r0u = 1.0 / e0u
    r0v = 1.0 / e0v
    r1u = 1.0 / e1u
    r1v = 1.0 / e1v
    eds = ((e0u, e0v, r0u, r0v), (e1u, e1v, r1u, r1v),
           (-e0u, -e0v, -r0u, -r0v), (-e1u, -e1v, -r1u, -r1v))
    area2 = jnp.zeros_like(x)
    for j in range(4):
        du, dv, ru, rv = eds[j]
        dt = _clip_dt(us[j], vs[j], du, dv, ru, rv, hx, hy)
        area2 = area2 + dt * (us[j] * dv - vs[j] * du)

    qs = []
    for qxj, qyj in ((qx0, qy0), (qx1, qy1), (qx2, qy2), (qx3, qy3)):
        tx = qxj - x
        ty = qyj - y
        qs.append((ca * tx + sa * ty, ca * ty - sa * tx))
    d0u = ca * d0x + sa * d0y
    d0v = ca * d0y - sa * d0x
    d1u = ca * d1x + sa * d1y
    d1v = ca * d1y - sa * d1x
    s0u = 1.0 / d0u
    s0v = 1.0 / d0v
    s1u = 1.0 / d1u
    s1v = 1.0 / d1v
    bds = ((d0u, d0v, s0u, s0v), (d1u, d1v, s1u, s1v),
           (-d0u, -d0v, -s0u, -s0v), (-d1u, -d1v, -s1u, -s1v))
    dtsum = jnp.zeros_like(x)
    for j in range(4):
        du, dv, ru, rv = bds[j]
        dtsum = dtsum + _clip_dt(qs[j][0], qs[j][1], du, dv, ru, rv, hxa, hya)
    area2 = area2 + dtsum * (2.0 * hx * hy)

    area = jnp.maximum(0.5 * area2, 0.0)
    h = jnp.maximum(jnp.minimum(za2, zb2) - jnp.maximum(za1, zb1), 0.0)
    inter = area * h
    iou = inter / jnp.maximum(va + vb - inter, 1e-6)
    upd = iou > best
    best = jnp.where(upd, iou, best)
    bki = jnp.where(upd, jnp.broadcast_to(k, bki.shape), bki)
    return best, bki


def _derive(x, y, z, dxa, dya, dza, r):
    ca = jnp.cos(r)
    sa = jnp.sin(r)
    hxa = 0.5 * dxa
    hya = 0.5 * dya
    cxh = ca * hxa
    sxh = sa * hxa
    cyh = ca * hya
    syh = sa * hya
    pax = (x + cxh - syh, x - cxh - syh, x - cxh + syh, x + cxh + syh)
    pay = (y + sxh + cyh, y - sxh + cyh, y - sxh - cyh, y + sxh - cyh)
    e = (-2.0 * cxh, -2.0 * sxh, 2.0 * syh, -2.0 * cyh)
    za1 = z - 0.5 * dza
    za2 = z + 0.5 * dza
    va = dxa * dya * dza
    return pax, pay, e, ca, sa, hxa, hya, za1, za2, va


def _tc_assign_body(pred_ref, gt_ref, out_ref):
    x = pred_ref[0, 0]
    y = pred_ref[0, 1]
    pax, pay, e, ca, sa, hxa, hya, za1, za2, va = _derive(
        x, y, pred_ref[0, 2], pred_ref[0, 3], pred_ref[0, 4],
        pred_ref[0, 5], pred_ref[0, 6])
    p = (pax[0], pax[1], pax[2], pax[3], pay[0], pay[1], pay[2], pay[3],
         e[0], e[1], e[2], e[3], x, y, ca, sa, hxa, hya, za1, za2, va)

    shape = x.shape
    best0 = jnp.full(shape, -1.0, jnp.float32)
    bki0 = jnp.zeros(shape, jnp.int32)

    def step(k, carry):
        best, bki = carry
        g = tuple(gt_ref[0, k, j] for j in range(21))
        return _pair_step(p, g, k, best, bki)

    best, bki = jax.lax.fori_loop(0, _K, step, (best0, bki0), unroll=50)
    out_ref[0] = jnp.where(best < _IOU_THRESHOLD, -1, bki)


def _prep_body(pred_ref, stat_ref, out_ref):
    x = pred_ref[0, 0]
    y = pred_ref[0, 1]
    pax, pay, e, ca, sa, hxa, hya, za1, za2, va = _derive(
        x, y, pred_ref[0, 2], pred_ref[0, 3], pred_ref[0, 4],
        pred_ref[0, 5], pred_ref[0, 6])
    for j in range(4):
        out_ref[0, j] = pax[j]
        out_ref[0, 4 + j] = pay[j]
        out_ref[0, 8 + j] = e[j]
    out_ref[0, 12] = x
    out_ref[0, 13] = y
    out_ref[0, 14] = ca
    out_ref[0, 15] = sa
    out_ref[0, 16] = hxa
    out_ref[0, 17] = hya
    out_ref[0, 18] = za1
    out_ref[0, 19] = za2
    out_ref[0, 20] = va
    out_ref[0, 21] = jnp.sqrt(hxa * hxa + hya * hya)
    out_ref[0, 22] = stat_ref[0, 0]
    out_ref[0, 23] = stat_ref[0, 1]
    out_ref[0, 24] = stat_ref[0, 2]
    zero = jnp.zeros_like(x)
    for j in range(25, _NP):
        out_ref[0, j] = zero


def _sc_assign_body(planes_hbm, gt_hbm, out_hbm, pv, gv, ov, bv, kv):
    cid = lax.axis_index("c")
    sid = lax.axis_index("s")
    wid = sid * 2 + cid
    st = wid // _NCHUNK
    ck = lax.rem(wid, _NCHUNK)
    pltpu.sync_copy(planes_hbm.at[st, ck], pv)
    pltpu.sync_copy(gt_hbm.at[st], gv)

    def outer(v, carry_o):
        off = v * 16
        p = tuple(pv[pl.ds(j * _CHUNK + off, 16)] for j in range(21))
        xlo = pv[pl.ds(22 * _CHUNK + off, 16)][0]
        xhi = pv[pl.ds(23 * _CHUNK + off, 16)][0]
        rmx = pv[pl.ds(24 * _CHUNK + off, 16)][0]
        xc = 0.5 * (xlo + xhi)
        xh = 0.5 * (xhi - xlo) + rmx
        bv[pl.ds(0, 16)] = jnp.full((16,), -1.0, jnp.float32)
        kv[pl.ds(0, 16)] = jnp.zeros((16,), jnp.int32)

        def inner(k, carry):
            cxs = gv[pl.ds(k * 384 + 0 * 16, 16)][0]
            rbs = gv[pl.ds(k * 384 + 21 * 16, 16)][0]
            d = cxs - xc

            @pl.when(jnp.abs(d) < xh + rbs)
            def _():
                g = tuple(gv[pl.ds(k * 384 + j * 16, 16)] for j in range(21))
                best, bki = _pair_step(p, g, k, bv[pl.ds(0, 16)],
                                       kv[pl.ds(0, 16)])
                bv[pl.ds(0, 16)] = best
                kv[pl.ds(0, 16)] = bki

            return carry

        lax.fori_loop(0, _K, inner, 0)
        best = bv[pl.ds(0, 16)]
        ov[pl.ds(off, 16)] = jnp.where(best < _IOU_THRESHOLD,
                                       jnp.full((16,), -1, jnp.int32),
                                       kv[pl.ds(0, 16)])
        return carry_o

    lax.fori_loop(0, _NV, outer, 0)
    pltpu.sync_copy(ov, out_hbm.at[st, ck])


def _gt_scalars(gt):
    cx = gt[..., 0]
    cy = gt[..., 1]
    zc = gt[..., 2]
    dx = gt[..., 3]
    dy = gt[..., 4]
    dz = gt[..., 5]
    rr = gt[..., 6]
    cb = jnp.cos(rr)
    sb = jnp.sin(rr)
    hx = 0.5 * dx
    hy = 0.5 * dy
    cxh = cb * hx
    sxh = sb * hx
    cyh = cb * hy
    syh = sb * hy
    qx0 = cx + cxh - syh
    qy0 = cy + sxh + cyh
    qx1 = cx - cxh - syh
    qy1 = cy - sxh + cyh
    qx2 = cx - cxh + syh
    qy2 = cy - sxh - cyh
    qx3 = cx + cxh + syh
    qy3 = cy + sxh - cyh
    d0x = qx1 - qx0
    d0y = qy1 - qy0
    d1x = qx2 - qx1
    d1y = qy2 - qy1
    zb1 = zc - 0.5 * dz
    zb2 = zc + 0.5 * dz
    vb = dx * dy * dz
    rb = jnp.sqrt(hx * hx + hy * hy)
    pad = jnp.zeros_like(cx)
    return jnp.stack([cx, cy, cb, sb, hx, hy, zb1, zb2, vb,
                      qx0, qy0, qx1, qy1, qx2, qy2, qx3, qy3,
                      d0x, d0y, d1x, d1y, rb, pad, pad], axis=-1)


def kernel(teacher_boxes, student_boxes, gt_boxes):
    B, N, _ = teacher_boxes.shape
    pred = jnp.concatenate([teacher_boxes, student_boxes], axis=0)
    padbox = jnp.zeros((2 * B, _NPAD - N, 7), jnp.float32).at[:, :, 3:6].set(1.0)
    pred = jnp.concatenate([pred, padbox], axis=1)

    gtp = _gt_scalars(gt_boxes)
    gtp = jnp.tile(gtp, (2, 1, 1))
    gtb = jnp.broadcast_to(gtp[..., None], (_NSETS, _K, 24, 16))
    gtb = gtb.reshape(_NSETS, _K * 24 * 16)

    preds = pred[:, _TC_N:]
    perm = jnp.argsort(preds[:, :, 0], axis=-1)
    preds = jnp.take_along_axis(preds, perm[:, :, None], axis=1)
    planes_s = jnp.transpose(preds, (0, 2, 1))
    planes_s = jnp.concatenate(
        [planes_s, jnp.zeros((_NSETS, 1, _SC_N), jnp.float32)], axis=1)
    rows_s = _SC_N // _LANES
    planes_s = planes_s.reshape(_NSETS, 8, rows_s, _LANES)

    xg = preds[:, :, 0].reshape(_NSETS, _SC_N // 16, 16)
    rg = 0.5 * jnp.sqrt(preds[:, :, 3] ** 2 + preds[:, :, 4] ** 2)
    rg = rg.reshape(_NSETS, _SC_N // 16, 16)
    stats = jnp.stack([xg.min(-1), xg.max(-1), rg.max(-1)], axis=1)
    stats = jnp.broadcast_to(stats[..., None], (_NSETS, 3, _SC_N // 16, 16))
    stats = stats.reshape(_NSETS, 3, rows_s, _LANES)
    stats = jnp.concatenate(
        [stats, jnp.zeros((_NSETS, 5, rows_s, _LANES), jnp.float32)], axis=1)

    derived = pl.pallas_call(
        _prep_body,
        grid=(_NSETS, rows_s // _RB),
        in_specs=[
            pl.BlockSpec((1, 8, _RB, _LANES), lambda c, nb: (c, 0, nb, 0)),
            pl.BlockSpec((1, 8, _RB, _LANES), lambda c, nb: (c, 0, nb, 0)),
        ],
        out_specs=pl.BlockSpec((1, _NP, _RB, _LANES), lambda c, nb: (c, 0, nb, 0)),
        out_shape=jax.ShapeDtypeStruct((_NSETS, _NP, rows_s, _LANES),
                                       jnp.float32),
    )(planes_s, stats)
    derived = derived.reshape(_NSETS, _NP, _NCHUNK, _CHUNK).transpose(0, 2, 1, 3)
    derived = derived.reshape(_NSETS, _NCHUNK, _NP * _CHUNK)

    sc = functools.partial(
        pl.kernel,
        out_type=jax.ShapeDtypeStruct((_NSETS, _NCHUNK, _CHUNK), jnp.int32),
        mesh=plsc.VectorSubcoreMesh(core_axis_name="c", subcore_axis_name="s"),
        scratch_types=[
            pltpu.VMEM((_NP * _CHUNK,), jnp.float32),
            pltpu.VMEM((_K * 24 * 16,), jnp.float32),
            pltpu.VMEM((_CHUNK,), jnp.int32),
            pltpu.VMEM((16,), jnp.float32),
            pltpu.VMEM((16,), jnp.int32),
        ],
    )(_sc_assign_body)
    out_s = sc(derived, gtb)
    out_s = out_s.reshape(_NSETS, _SC_N)
    out_s = jnp.zeros_like(out_s).at[
        jnp.arange(_NSETS)[:, None], perm].set(out_s)

    planes_t = jnp.transpose(pred[:, :_TC_N], (0, 2, 1))
    planes_t = jnp.concatenate(
        [planes_t, jnp.zeros((_NSETS, 1, _TC_N), jnp.float32)], axis=1)
    planes_t = planes_t.reshape(_NSETS, 8, _TC_N // _LANES, _LANES)

    out_t = pl.pallas_call(
        _tc_assign_body,
        grid=(_NSETS, _TC_NBLK),
        in_specs=[
            pl.BlockSpec((1, 8, _RB, _LANES), lambda c, nb: (c, 0, nb, 0)),
            pl.BlockSpec((1, _K, 24), lambda c, nb: (c, 0, 0),
                         memory_space=pltpu.SMEM),
        ],
        out_specs=pl.BlockSpec((1, _RB, _LANES), lambda c, nb: (c, nb, 0)),
        out_shape=jax.ShapeDtypeStruct((_NSETS, _TC_N // _LANES, _LANES),
                                       jnp.int32),
    )(planes_t, gtp)
    out_t = out_t.reshape(_NSETS, _TC_N)

    out = jnp.concatenate([out_t, out_s], axis=1)[:, :N]
    return out[:B], out[B:]
